# Initial kernel scaffold; baseline (speedup 1.0000x reference)
import jax
import jax.numpy as jnp
from jax.experimental import pallas as pl


def kernel(features, edge_index, W):
    raise NotImplementedError("write your pallas kernel here")


# speedup vs baseline: 3.7079x; 3.7079x over previous
import functools

import jax
import jax.numpy as jnp
from jax import lax
from jax.experimental import pallas as pl
from jax.experimental.pallas import tpu as pltpu
import jax.experimental.pallas.tpu_sc as plsc

N = 10000
E = 320000
F = 128
L = 16
NC = 2
NS = 16
NW = NC * NS
CH = 128
CHUNKS = 80
EPAD = NW * CHUNKS * CH
NACC = NS * 632
CROWS = NACC // CH
RB = 1000


def _tc_matmul_body(f_ref, w_ref, y_ref):
    y_ref[...] = jnp.dot(f_ref[...], w_ref[...],
                         preferred_element_type=jnp.float32)


def _tc_combine_body(y_ref, p0_ref, p1_ref, c_ref, out_ref):
    ssum = p0_ref[...] + p1_ref[...]
    cnt = jnp.sum(c_ref[...], axis=1)
    mean = ssum / jnp.maximum(cnt, 1.0)[:, None]
    out_ref[:, :F] = jnp.maximum(y_ref[...], 0.0)
    out_ref[:, F:] = jnp.maximum(mean, 0.0)


def _sc_body(yext, srcs, dsts, zsum, zcnt, out_sum, out_cnt,
             src_v, dst_v, rows_v, hist_v, acc_sh, sem):
    c = lax.axis_index("c")
    s = lax.axis_index("s")
    wid = s * NC + c
    base = wid * (CHUNKS * CH)

    pltpu.sync_copy(zsum.at[pl.ds(s * 632, 632)], acc_sh.at[pl.ds(s * 632, 632)])
    pltpu.sync_copy(zcnt, hist_v)

    plsc.subcore_barrier()

    ones16 = jnp.full((L,), 1.0, jnp.float32)

    def chunk(j, carry):
        off = base + j * CH
        pltpu.sync_copy(srcs.at[pl.ds(off, CH)], src_v)
        pltpu.sync_copy(dsts.at[pl.ds(off, CH)], dst_v)
        pltpu.async_copy(yext.at[src_v], rows_v, sem).wait()
        pltpu.sync_copy(rows_v, acc_sh.at[dst_v], add=True)
        for k in range(CH // L):
            idx = dst_v[pl.ds(k * L, L)]
            plsc.addupdate_scatter(hist_v, [idx], ones16)
        return carry

    lax.fori_loop(0, CHUNKS, chunk, 0)

    pltpu.sync_copy(hist_v, out_cnt.at[wid])

    plsc.subcore_barrier()

    @pl.when(s < NS - 1)
    def _():
        pltpu.sync_copy(acc_sh.at[pl.ds(s * 632, 632)],
                        out_sum.at[pl.ds(c * N + s * 632, 632)])

    @pl.when(s == NS - 1)
    def _():
        pltpu.sync_copy(acc_sh.at[pl.ds((NS - 1) * 632, 520)],
                        out_sum.at[pl.ds(c * N + (NS - 1) * 632, 520)])


_sc_segsum = functools.partial(
    pl.kernel,
    out_type=(jax.ShapeDtypeStruct((NC * N, F), jnp.float32),
              jax.ShapeDtypeStruct((NW, NACC), jnp.float32)),
    mesh=plsc.VectorSubcoreMesh(core_axis_name="c", subcore_axis_name="s",
                                num_cores=NC, num_subcores=NS),
    compiler_params=pltpu.CompilerParams(needs_layout_passes=False),
    scratch_types=[
        pltpu.VMEM((CH,), jnp.int32),
        pltpu.VMEM((CH,), jnp.int32),
        pltpu.VMEM((CH, F), jnp.float32),
        pltpu.VMEM((NACC,), jnp.float32),
        pltpu.VMEM_SHARED((NACC, F), jnp.float32),
        pltpu.SemaphoreType.DMA,
    ],
)(_sc_body)


@jax.jit
def kernel(features, edge_index, W):
    src = edge_index[1]
    dst = edge_index[0]
    pad = EPAD - E
    srcs = jnp.concatenate([src, jnp.zeros((pad,), jnp.int32)])
    dsts = jnp.concatenate([dst, jnp.full((pad,), N, jnp.int32)])
    zsum = jnp.zeros((NACC, F), jnp.float32)
    zcnt = jnp.zeros((NACC,), jnp.float32)

    yext = pl.pallas_call(
        _tc_matmul_body,
        grid=(N // RB,),
        in_specs=[
            pl.BlockSpec((RB, F), lambda i: (i, 0)),
            pl.BlockSpec((F, F), lambda i: (0, 0)),
        ],
        out_specs=pl.BlockSpec((RB, F), lambda i: (i, 0)),
        out_shape=jax.ShapeDtypeStruct((N, F), jnp.float32),
    )(features, W)

    psum, pcnt = _sc_segsum(yext, srcs, dsts, zsum, zcnt)
    pcnt_t = pcnt.T[:N]

    nb = N // RB

    out = pl.pallas_call(
        _tc_combine_body,
        grid=(nb,),
        in_specs=[
            pl.BlockSpec((RB, F), lambda i: (i, 0)),
            pl.BlockSpec((RB, F), lambda i: (i, 0)),
            pl.BlockSpec((RB, F), lambda i: (i + nb, 0)),
            pl.BlockSpec((RB, NW), lambda i: (i, 0)),
        ],
        out_specs=pl.BlockSpec((RB, 2 * F), lambda i: (i, 0)),
        out_shape=jax.ShapeDtypeStruct((N, 2 * F), jnp.float32),
    )(yext, psum, psum, pcnt_t)

    return out
